# baseline (speedup 1.0000x reference)
import functools

import jax
import jax.numpy as jnp
from jax import lax
from jax.experimental import pallas as pl
from jax.experimental.pallas import tpu as pltpu
from jax.experimental.pallas import tpu_sc as plsc

RANK = 16
LANES = 16
NUM_WORKERS = 32
CHUNK = 800


def _tatd_kernel(nnz, ndim):
    num_chunks = nnz // CHUNK
    assert num_chunks * CHUNK == nnz and CHUNK % LANES == 0
    groups = CHUNK // LANES
    base_n = num_chunks // NUM_WORKERS
    rem = num_chunks % NUM_WORKERS
    max_n = base_n + (1 if rem else 0)
    outer_iters = (max_n + 1) // 2

    mesh = plsc.VectorSubcoreMesh(core_axis_name="c", subcore_axis_name="s")

    @functools.partial(
        pl.kernel,
        mesh=mesh,
        compiler_params=pltpu.CompilerParams(
            needs_layout_passes=False, use_tc_tiling_on_sc=False),
        out_type=jax.ShapeDtypeStruct((nnz,), jnp.float32),
        scratch_types=[
            pltpu.VMEM((CHUNK,), jnp.int32),
            pltpu.VMEM((CHUNK,), jnp.int32),
            pltpu.VMEM((CHUNK,), jnp.int32),
            pltpu.VMEM((CHUNK, RANK), jnp.float32),
            pltpu.VMEM((CHUNK, RANK), jnp.float32),
            pltpu.VMEM((CHUNK, RANK), jnp.float32),
            pltpu.VMEM((CHUNK, RANK), jnp.float32),
            pltpu.VMEM((CHUNK, RANK), jnp.float32),
            pltpu.VMEM((CHUNK, RANK), jnp.float32),
            pltpu.VMEM((CHUNK,), jnp.float32),
            pltpu.VMEM((CHUNK,), jnp.float32),
            pltpu.SemaphoreType.DMA,
            pltpu.SemaphoreType.DMA,
            pltpu.SemaphoreType.DMA,
            pltpu.SemaphoreType.DMA,
        ],
    )
    def k(idx_hbm, f0_hbm, f1_hbm, f2_hbm, out_hbm,
          idx0_v, idx1_v, idx2_v,
          r0a, r1a, r2a, r0b, r1b, r2b,
          out_a, out_b,
          sem_ga, sem_gb, sem_oa, sem_ob):
        wid = lax.axis_index("s") * 2 + lax.axis_index("c")
        extra = jnp.minimum(wid, rem)
        lo = wid * base_n + extra
        n = base_n + jnp.where(wid < rem, 1, 0)
        lane = lax.iota(jnp.int32, LANES)
        rows = ((r0a, r1a, r2a), (r0b, r1b, r2b))
        outs = (out_a, out_b)
        sems_g = (sem_ga, sem_gb)
        sems_o = (sem_oa, sem_ob)
        fs = (f0_hbm, f1_hbm, f2_hbm)
        idxs = (idx0_v, idx1_v, idx2_v)

        def load_idx(chunk_id):
            base = chunk_id * CHUNK
            for m in range(3):
                pltpu.sync_copy(idx_hbm.at[pl.ds(m * nnz + base, CHUNK)],
                                idxs[m])

        def issue_gathers(b):
            for m in range(3):
                pltpu.async_copy(fs[m].at[idxs[m]], rows[b][m], sems_g[b])

        def wait_gathers(b):
            for m in range(3):
                pltpu.make_async_copy(fs[m].at[idxs[m]], rows[b][m],
                                      sems_g[b]).wait()

        def compute(b):
            r0, r1, r2 = rows[b]
            out_v = outs[b]

            @plsc.parallel_loop(0, groups)
            def group_body(g):
                row_ids = g * LANES + lane
                acc = jnp.zeros((LANES,), jnp.float32)
                for r in range(RANK):
                    col = jnp.full((LANES,), r, jnp.int32)
                    v0 = plsc.load_gather(r0, [row_ids, col])
                    v1 = plsc.load_gather(r1, [row_ids, col])
                    v2 = plsc.load_gather(r2, [row_ids, col])
                    acc = acc + v0 * v1 * v2
                out_v[pl.ds(g * LANES, LANES)] = acc

        def issue_out(kk, b):
            base = (lo + kk) * CHUNK
            pltpu.async_copy(outs[b], out_hbm.at[pl.ds(base, CHUNK)],
                             sems_o[b])

        def wait_out(b):
            pltpu.make_async_copy(outs[b], out_hbm.at[pl.ds(0, CHUNK)],
                                  sems_o[b]).wait()

        load_idx(lo)
        issue_gathers(0)

        def body(kk, b):
            wait_gathers(b)

            @pl.when(kk + 1 < n)
            def _():
                load_idx(lo + kk + 1)
                issue_gathers(1 - b)

            @pl.when(kk >= 2)
            def _():
                wait_out(b)

            compute(b)
            issue_out(kk, b)

        def outer(i, _):
            kk = i * 2

            @pl.when(kk < n)
            def _():
                body(kk, 0)

            @pl.when(kk + 1 < n)
            def _():
                body(kk + 1, 1)

            return 0

        lax.fori_loop(0, outer_iters, outer, 0)

        wait_out((0))
        wait_out((1))

    return k


def kernel(indices_list, f0, f1, f2):
    nnz = indices_list.shape[1]
    ndim = f0.shape[0]
    idx_flat = indices_list.astype(jnp.int32).reshape(-1)
    return _tatd_kernel(nnz, ndim)(idx_flat, f0, f1, f2)
